# baseline (speedup 1.0000x reference)
import functools

import jax
import jax.numpy as jnp
from jax.experimental import pallas as pl
from jax.experimental.pallas import tpu as pltpu

_N = 10000
_E = 2000
_D = 128
_CH = 1000
_DN = (((0,), (0,)), ((), ()))
_DNAT = (((1,), (0,)), ((), ()))
_F32 = jnp.float32


def _split_bf16(v):
    hi = v.astype(jnp.bfloat16)
    lo = (v - hi.astype(_F32)).astype(jnp.bfloat16)
    return hi, lo


def _hypersage_kernel(x_ref, inc_ref, w1_ref, w2_ref, wlin_ref, blin_ref,
                      out_ref, feat, edge, ehi, elo, ide):
    nchunks = _N // _CH

    ones_e = jnp.ones((_E, 1), jnp.bfloat16)
    ones_c = jnp.ones((_CH, 1), jnp.bfloat16)
    ide[...] = jnp.zeros((_E, 1), _F32)

    def dege_body(i, _):
        r = pl.ds(i * _CH, _CH)
        ide[...] += jax.lax.dot_general(inc_ref[r, :], ones_c, _DN,
                                        preferred_element_type=_F32)
        return 0

    jax.lax.fori_loop(0, nchunks, dege_body, 0)
    ide[...] = 1.0 / ide[...]

    def layer(in_ref, W, out_feat_ref):
        edge[...] = jnp.zeros((_E, _D), _F32)

        def edge_body(i, _):
            r = pl.ds(i * _CH, _CH)
            f = in_ref[r, :]
            hi, lo = _split_bf16(f * f)
            inc_c = inc_ref[r, :]
            acc = jax.lax.dot_general(inc_c, hi, _DN,
                                      preferred_element_type=_F32)
            acc += jax.lax.dot_general(inc_c, lo, _DN,
                                       preferred_element_type=_F32)
            edge[...] += acc
            return 0

        jax.lax.fori_loop(0, nchunks, edge_body, 0)
        h_, l_ = _split_bf16(edge[...] * ide[...])
        ehi[...] = h_
        elo[...] = l_

        def node_body(i, _):
            r = pl.ds(i * _CH, _CH)
            inc_c = inc_ref[r, :]
            t = jax.lax.dot_general(inc_c, ehi[...], _DNAT,
                                    preferred_element_type=_F32)
            t += jax.lax.dot_general(inc_c, elo[...], _DNAT,
                                     preferred_element_type=_F32)
            deg_c = jax.lax.dot_general(inc_c, ones_e, _DNAT,
                                        preferred_element_type=_F32)
            mv = jnp.sqrt(t / deg_c)
            h = jax.lax.dot_general(mv, W, _DNAT, preferred_element_type=_F32)
            h = jnp.maximum(h, 0.0)
            norm = jnp.sqrt(jnp.sum(h * h, axis=-1, keepdims=True))
            out_feat_ref[r, :] = h / (norm + 1e-12)
            return 0

        jax.lax.fori_loop(0, nchunks, node_body, 0)

    layer(x_ref, w1_ref[...], feat)
    layer(feat, w2_ref[...], feat)

    def max_body(i, acc):
        r = pl.ds(i * _CH, _CH)
        return jnp.maximum(acc, jnp.max(feat[r, :], axis=0, keepdims=True))

    pooled = jax.lax.fori_loop(
        0, nchunks, max_body, jnp.full((1, _D), -jnp.inf, _F32))
    dn_t = (((1,), (1,)), ((), ()))
    out_ref[...] = (
        jax.lax.dot_general(pooled, wlin_ref[...], dn_t,
                            preferred_element_type=_F32)
        + blin_ref[...])


@jax.jit
def kernel(x_0, incidence, W1, W2, Wlin, b_lin):
    inc_bf16 = incidence.astype(jnp.bfloat16)
    out = pl.pallas_call(
        _hypersage_kernel,
        out_shape=jax.ShapeDtypeStruct((1, Wlin.shape[0]), jnp.float32),
        scratch_shapes=[
            pltpu.VMEM((_N, _D), _F32),
            pltpu.VMEM((_E, _D), _F32),
            pltpu.VMEM((_E, _D), jnp.bfloat16),
            pltpu.VMEM((_E, _D), jnp.bfloat16),
            pltpu.VMEM((_E, 1), _F32),
        ],
        compiler_params=pltpu.CompilerParams(
            vmem_limit_bytes=64 * 1024 * 1024,
        ),
    )(x_0, inc_bf16, W1, W2, Wlin, b_lin.reshape(1, -1))
    return out.reshape(-1)


# speedup vs baseline: 1.3803x; 1.3803x over previous
import functools

import jax
import jax.numpy as jnp
from jax.experimental import pallas as pl
from jax.experimental.pallas import tpu as pltpu

_N = 10000
_E = 2000
_D = 128
_CH = 1000
_DN = (((0,), (0,)), ((), ()))
_DNAT = (((1,), (0,)), ((), ()))
_F32 = jnp.float32


def _hypersage_kernel(x_ref, inc_ref, w1_ref, w2_ref, wlin_ref, blin_ref,
                      out_ref, feat, edge, ehi, ide):
    nchunks = _N // _CH

    ones_c = jnp.ones((_CH, 1), jnp.bfloat16)
    ide[...] = jnp.zeros((_E, 1), _F32)

    def dege_body(i, _):
        r = pl.ds(i * _CH, _CH)
        ide[...] += jax.lax.dot_general(inc_ref[r, :], ones_c, _DN,
                                        preferred_element_type=_F32)
        return 0

    jax.lax.fori_loop(0, nchunks, dege_body, 0)
    ide[...] = 1.0 / ide[...]

    def layer(in_ref, W, out_feat_ref):
        edge[...] = jnp.zeros((_E, _D), _F32)

        def edge_body(i, _):
            r = pl.ds(i * _CH, _CH)
            f = in_ref[r, :]
            hi = (f * f).astype(jnp.bfloat16)
            inc_c = inc_ref[r, :]
            edge[...] += jax.lax.dot_general(inc_c, hi, _DN,
                                             preferred_element_type=_F32)
            return 0

        jax.lax.fori_loop(0, nchunks, edge_body, 0)
        ehi[...] = (edge[...] * ide[...]).astype(jnp.bfloat16)

        def node_body(i, _):
            r = pl.ds(i * _CH, _CH)
            inc_c = inc_ref[r, :]
            t = jax.lax.dot_general(inc_c, ehi[...], _DNAT,
                                    preferred_element_type=_F32)
            mv = jnp.sqrt(t)
            h = jax.lax.dot_general(mv, W, _DNAT, preferred_element_type=_F32)
            h = jnp.maximum(h, 0.0)
            norm = jnp.sqrt(jnp.sum(h * h, axis=-1, keepdims=True))
            out_feat_ref[r, :] = h / (norm + 1e-12)
            return 0

        jax.lax.fori_loop(0, nchunks, node_body, 0)

    layer(x_ref, w1_ref[...], feat)
    layer(feat, w2_ref[...], feat)

    def max_body(i, acc):
        r = pl.ds(i * _CH, _CH)
        return jnp.maximum(acc, jnp.max(feat[r, :], axis=0, keepdims=True))

    pooled = jax.lax.fori_loop(
        0, nchunks, max_body, jnp.full((1, _D), -jnp.inf, _F32))
    dn_t = (((1,), (1,)), ((), ()))
    out_ref[...] = (
        jax.lax.dot_general(pooled, wlin_ref[...], dn_t,
                            preferred_element_type=_F32)
        + blin_ref[...])


@jax.jit
def kernel(x_0, incidence, W1, W2, Wlin, b_lin):
    inc_bf16 = incidence.astype(jnp.bfloat16)
    out = pl.pallas_call(
        _hypersage_kernel,
        out_shape=jax.ShapeDtypeStruct((1, Wlin.shape[0]), jnp.float32),
        scratch_shapes=[
            pltpu.VMEM((_N, _D), _F32),
            pltpu.VMEM((_E, _D), _F32),
            pltpu.VMEM((_E, _D), jnp.bfloat16),
            pltpu.VMEM((_E, 1), _F32),
        ],
        compiler_params=pltpu.CompilerParams(
            vmem_limit_bytes=64 * 1024 * 1024,
        ),
    )(x_0, inc_bf16, W1, W2, Wlin, b_lin.reshape(1, -1))
    return out.reshape(-1)
